# baseline (speedup 1.0000x reference)
import functools

import jax
import jax.numpy as jnp
from jax import lax
from jax.experimental import pallas as pl
from jax.experimental.pallas import tpu as pltpu
from jax.experimental.pallas import tpu_sc as plsc

NC = 2
NS = 16
NW = NC * NS
L = 16
LP = 128


def _emb_kernel_fn(B, S, D, CH, GB):
    G = CH // GB
    b_per_w = B // NW
    n_chunks = b_per_w // CH
    n_pairs = n_chunks // 2
    seqs_per_chunk = CH // S
    DL = D // L

    def body(tok_hbm, e_hbm, p_hbm, out_hbm,
             p_v, idx0, idx1, grow0, grow1, orow0, orow1,
             gsem0, gsem1, osem0, osem1):
        cid = lax.axis_index("c")
        sid = lax.axis_index("s")
        wid = sid * NC + cid
        base = wid * b_per_w

        pltpu.sync_copy(p_hbm, p_v)

        def fire(ci, idx_v, grow_v, sem):
            rbase = base + ci * CH
            pltpu.sync_copy(tok_hbm.at[pl.ds(rbase, CH)], idx_v)
            for j in range(G):
                pltpu.async_copy(
                    e_hbm.at[idx_v.at[pl.ds(j * GB, GB)]],
                    grow_v.at[pl.ds(j * GB, GB)],
                    sem,
                )

        def drain_gathers(idx_v, grow_v, sem):
            for j in range(G):
                pltpu.make_async_copy(
                    e_hbm.at[idx_v.at[pl.ds(j * GB, GB)]],
                    grow_v.at[pl.ds(j * GB, GB)],
                    sem,
                ).wait()

        def add_chunk(grow_v, orow_v):
            def add_row(r, c2):
                pv = [p_v[r, pl.ds(k * L, L)] for k in range(DL)]
                for sq in range(seqs_per_chunk):
                    row = sq * S + r
                    for k in range(DL):
                        sl = pl.ds(k * L, L)
                        orow_v[row, sl] = grow_v[row, sl] + pv[k]
                return c2

            lax.fori_loop(0, S, add_row, 0)

        SPC = CH // S

        def write(ci, orow_v, sem):
            b0 = (base + ci * CH) // S
            for sq in range(SPC):
                pltpu.async_copy(
                    orow_v.at[pl.ds(sq * S, S)],
                    out_hbm.at[b0 + sq],
                    sem,
                )

        def drain_write(orow_v, sem):
            for sq in range(SPC):
                pltpu.make_async_copy(
                    orow_v.at[pl.ds(sq * S, S)],
                    out_hbm.at[sq],
                    sem,
                ).wait()

        fire(0, idx0, grow0, gsem0)

        def pair_body(it, carry):
            a = 2 * it
            b = a + 1

            @pl.when(it > 0)
            def _():
                drain_write(orow1, osem1)

            fire(b, idx1, grow1, gsem1)

            drain_gathers(idx0, grow0, gsem0)
            add_chunk(grow0, orow0)
            write(a, orow0, osem0)

            @pl.when(it < n_pairs - 1)
            def _():
                drain_write(orow0, osem0)
                fire(a + 2, idx0, grow0, gsem0)

            drain_gathers(idx1, grow1, gsem1)
            add_chunk(grow1, orow1)
            write(b, orow1, osem1)
            return carry

        lax.fori_loop(0, n_pairs, pair_body, 0)
        drain_write(orow0, osem0)
        drain_write(orow1, osem1)

    return body


@functools.lru_cache(maxsize=None)
def _make_emb_lookup(B, S, D, CH, GB):
    mesh = plsc.VectorSubcoreMesh(core_axis_name="c", subcore_axis_name="s")
    body = _emb_kernel_fn(B, S, D, CH, GB)
    return pl.kernel(
        body,
        out_type=jax.ShapeDtypeStruct((B // S, S, D), jnp.float32),
        mesh=mesh,
        scratch_types=[
            pltpu.VMEM((S, D), jnp.float32),
            pltpu.VMEM((CH,), jnp.int32),
            pltpu.VMEM((CH,), jnp.int32),
            pltpu.VMEM((CH, LP), jnp.float32),
            pltpu.VMEM((CH, LP), jnp.float32),
            pltpu.VMEM((CH, D), jnp.float32),
            pltpu.VMEM((CH, D), jnp.float32),
            pltpu.SemaphoreType.DMA,
            pltpu.SemaphoreType.DMA,
            pltpu.SemaphoreType.DMA,
            pltpu.SemaphoreType.DMA,
        ],
        compiler_params=pltpu.CompilerParams(use_tc_tiling_on_sc=True),
    )


def kernel(tokens, E, P):
    batch, seq = tokens.shape
    _, d = E.shape
    B = batch * seq
    tok_flat = tokens.reshape(B).astype(jnp.int32)
    e_pad = jnp.pad(E, ((0, 0), (0, LP - d)))
    CH = seq
    GB = 40
    fn = _make_emb_lookup(B, seq, d, CH, GB)
    return fn(tok_flat, e_pad, P)
